# Initial kernel scaffold; baseline (speedup 1.0000x reference)
import jax
import jax.numpy as jnp
from jax.experimental import pallas as pl


def kernel(x, edge_index, Wl1, bl1, Wr1, Wl2, bl2, Wr2):
    raise NotImplementedError("write your pallas kernel here")


# speedup vs baseline: 2.2829x; 2.2829x over previous
import dataclasses

import jax
import jax.numpy as jnp
from jax import lax
from jax.experimental import pallas as pl
from jax.experimental.pallas import tpu as pltpu
from jax.experimental.pallas import tpu_sc as plsc

N = 10000
E = 320000
D = 128

NC = 2
NS = 16
NW = NC * NS
EW = E // NW
P = 320
NPAD = NW * P
ROWPAD = P + 8
TRASH = P + 7
REGW = EW + 752
OFFW = 40
SENT = (1 << 14) - 1
CHUNK = 128

_OMUL = 6554
_OSHR = 21

_mesh = plsc.VectorSubcoreMesh(core_axis_name="c", subcore_axis_name="s")

_sc_params = pltpu.CompilerParams()
if "needs_layout_passes" in pltpu.CompilerParams.__dataclass_fields__:
    _sc_params = dataclasses.replace(_sc_params, needs_layout_passes=False)


def _wid():
    return lax.axis_index("s") * NC + lax.axis_index("c")


def _route_body(ei_hbm, edges_hbm, offs_hbm, src_v, dst_v, ebuf, cnts, curs,
                offv, sem):
    w = _wid()
    base = w * EW
    pltpu.async_copy(ei_hbm.at[pl.ds(base, EW)], src_v, sem).wait()
    pltpu.async_copy(ei_hbm.at[pl.ds(E + base, EW)], dst_v, sem).wait()

    zeros = jnp.zeros((16,), jnp.int32)
    cnts[pl.ds(0, 16)] = zeros
    cnts[pl.ds(16, 16)] = zeros

    @pl.loop(0, EW // 16)
    def _(i):
        d = dst_v[pl.ds(i * 16, 16)]
        o = lax.shift_right_logical(d * _OMUL, _OSHR)
        cnt, last = plsc.scan_count(o)
        plsc.addupdate_scatter(cnts, [o], cnt, mask=last)

    c0 = cnts[pl.ds(0, 16)]
    c1 = cnts[pl.ds(16, 16)]
    p0 = jnp.bitwise_and(c0 + 7, -8)
    p1 = jnp.bitwise_and(c1 + 7, -8)
    i0 = plsc.cumsum(p0)
    s0 = jnp.sum(p0)
    i1 = plsc.cumsum(p1) + s0
    e0 = i0 - p0
    e1 = i1 - p1
    offv[pl.ds(0, 16)] = e0
    offv[pl.ds(16, 16)] = e1
    total = jnp.sum(p1) + s0
    offv[pl.ds(32, 16)] = jnp.full((16,), total, jnp.int32)
    curs[pl.ds(0, 16)] = e0
    curs[pl.ds(16, 16)] = e1

    @pl.loop(0, REGW // 16)
    def _(i):
        ebuf[pl.ds(i * 16, 16)] = jnp.full((16,), SENT, jnp.int32)

    @pl.loop(0, EW // 16)
    def _(i):
        s = src_v[pl.ds(i * 16, 16)]
        d = dst_v[pl.ds(i * 16, 16)]
        o = lax.shift_right_logical(d * _OMUL, _OSHR)
        cnt, last = plsc.scan_count(o)
        bpos = plsc.load_gather(curs, [o])
        pos = bpos + cnt - 1
        packed = jnp.bitwise_or(lax.shift_left(s, 14), d)
        plsc.store_scatter(ebuf, [pos], packed)
        plsc.addupdate_scatter(curs, [o], cnt, mask=last)

    pltpu.async_copy(ebuf, edges_hbm.at[pl.ds(w * REGW, REGW)], sem).wait()
    pltpu.async_copy(offv.at[pl.ds(0, OFFW)],
                     offs_hbm.at[pl.ds(w * OFFW, OFFW)], sem).wait()


@jax.jit
def _route(edge_index):
    kern = pl.kernel(
        _route_body,
        out_type=(
            jax.ShapeDtypeStruct((NW * REGW,), jnp.int32),
            jax.ShapeDtypeStruct((NW * OFFW,), jnp.int32),
        ),
        mesh=_mesh,
        compiler_params=_sc_params,
        scratch_types=[
            pltpu.VMEM((EW,), jnp.int32),
            pltpu.VMEM((EW,), jnp.int32),
            pltpu.VMEM((REGW,), jnp.int32),
            pltpu.VMEM((32,), jnp.int32),
            pltpu.VMEM((32,), jnp.int32),
            pltpu.VMEM((48,), jnp.int32),
            pltpu.SemaphoreType.DMA,
        ],
    )
    return kern(edge_index.reshape(2 * E))


def _agg_body(feat_hbm, edges_hbm, offs_hbm, agg_hbm, acc, rows, ebuf, sidx,
              dlv, offs_v, sem):
    b = _wid()
    pltpu.async_copy(offs_hbm, offs_v, sem).wait()

    neg = jnp.full((16,), -jnp.inf, jnp.float32)

    @pl.loop(0, ROWPAD * 8)
    def _(r):
        acc[pl.ds(r * 16, 16)] = neg

    blo = b * P
    iota = lax.iota(jnp.int32, 16)

    @pl.loop(0, NW)
    def _(w):
        start = jnp.max(plsc.load_gather(offs_v, [jnp.full((16,), w * OFFW, jnp.int32) + b]))
        end = jnp.max(plsc.load_gather(offs_v, [jnp.full((16,), w * OFFW + 1, jnp.int32) + b]))

        def chunk_step(ci, _):
            cbase = start + ci * CHUNK
            nrem = end - cbase
            rbase = pl.multiple_of(w * REGW + cbase, 8)
            pltpu.async_copy(edges_hbm.at[pl.ds(rbase, CHUNK)], ebuf,
                             sem).wait()
            for j in range(CHUNK // 16):
                e = ebuf[pl.ds(16 * j, 16)]
                sidx[pl.ds(16 * j, 16)] = lax.shift_right_logical(e, 14)
                dl = jnp.bitwise_and(e, SENT) - blo
                pos = iota + (16 * j)
                ok = jnp.logical_and(pos < nrem,
                                     jnp.logical_and(dl >= 0, dl < P))
                dlv[pl.ds(16 * j, 16)] = jnp.where(ok, dl, TRASH)
            pltpu.async_copy(feat_hbm.at[sidx], rows, sem).wait()

            @pl.loop(0, CHUNK)
            def _(j):
                dlb = plsc.load_gather(dlv, [jnp.full((16,), j, jnp.int32)])
                abase = dlb * 128 + iota
                for v in range(8):
                    addr = abase + (16 * v)
                    a = plsc.load_gather(acc, [addr])
                    r = rows[j, pl.ds(16 * v, 16)]
                    plsc.store_scatter(acc, [addr], jnp.maximum(a, r))
            return 0

        nchunks = lax.div(end - start + (CHUNK - 1), CHUNK)
        lax.fori_loop(0, nchunks, chunk_step, 0)

    pltpu.async_copy(acc.at[pl.ds(0, P * D)],
                     agg_hbm.at[pl.ds(blo * D, P * D)], sem).wait()


@jax.jit
def _aggregate(feat, edges, offs):
    kern = pl.kernel(
        _agg_body,
        out_type=jax.ShapeDtypeStruct((NPAD * D,), jnp.float32),
        mesh=_mesh,
        compiler_params=_sc_params,
        scratch_types=[
            pltpu.VMEM((ROWPAD * D,), jnp.float32),
            pltpu.VMEM((CHUNK, D), jnp.float32),
            pltpu.VMEM((CHUNK,), jnp.int32),
            pltpu.VMEM((CHUNK,), jnp.int32),
            pltpu.VMEM((CHUNK,), jnp.int32),
            pltpu.VMEM((NW * OFFW,), jnp.int32),
            pltpu.SemaphoreType.DMA,
        ],
    )
    return kern(feat, edges, offs).reshape(NPAD, D)


_ROWS = 1000


def _dense_body(a_ref, x_ref, wlT_ref, bl_ref, wrT_ref, o_ref):
    a = a_ref[...]
    a = jnp.where(jnp.isneginf(a), 0.0, a)
    acc = jnp.dot(a, wlT_ref[...], preferred_element_type=jnp.float32)
    acc += jnp.dot(x_ref[...], wrT_ref[...], preferred_element_type=jnp.float32)
    acc += bl_ref[...]
    o_ref[...] = jnp.where(acc >= 0, acc, 0.01 * acc)


def _dense(agg, x, Wl, bl, Wr):
    grid = (N // _ROWS,)
    return pl.pallas_call(
        _dense_body,
        grid=grid,
        in_specs=[
            pl.BlockSpec((_ROWS, D), lambda i: (i, 0)),
            pl.BlockSpec((_ROWS, D), lambda i: (i, 0)),
            pl.BlockSpec((D, D), lambda i: (0, 0)),
            pl.BlockSpec((1, D), lambda i: (0, 0)),
            pl.BlockSpec((D, D), lambda i: (0, 0)),
        ],
        out_specs=pl.BlockSpec((_ROWS, D), lambda i: (i, 0)),
        out_shape=jax.ShapeDtypeStruct((N, D), jnp.float32),
    )(agg, x, Wl.T, bl.reshape(1, D), Wr.T)


def kernel(x, edge_index, Wl1, bl1, Wr1, Wl2, bl2, Wr2):
    edges, offs = _route(edge_index)
    agg1 = _aggregate(x, edges, offs)
    h = _dense(agg1, x, Wl1, bl1, Wr1)
    agg2 = _aggregate(h, edges, offs)
    return _dense(agg2, h, Wl2, bl2, Wr2)


# speedup vs baseline: 2.3652x; 1.0360x over previous
import dataclasses

import jax
import jax.numpy as jnp
from jax import lax
from jax.experimental import pallas as pl
from jax.experimental.pallas import tpu as pltpu
from jax.experimental.pallas import tpu_sc as plsc

N = 10000
E = 320000
D = 128

NC = 2
NS = 16
NW = NC * NS
EW = E // NW
P = 320
NPAD = NW * P
ROWPAD = P + 8
TRASH = P + 7
REGW = EW + 752
OFFW = 40
SENT = (1 << 14) - 1
CHUNK = 128

_OMUL = 6554
_OSHR = 21

_mesh = plsc.VectorSubcoreMesh(core_axis_name="c", subcore_axis_name="s")

_sc_params = pltpu.CompilerParams()
if "needs_layout_passes" in pltpu.CompilerParams.__dataclass_fields__:
    _sc_params = dataclasses.replace(_sc_params, needs_layout_passes=False)


def _wid():
    return lax.axis_index("s") * NC + lax.axis_index("c")


def _route_body(ei_hbm, edges_hbm, offs_hbm, src_v, dst_v, ebuf, cnts, curs,
                offv, sem):
    w = _wid()
    base = w * EW
    pltpu.async_copy(ei_hbm.at[pl.ds(base, EW)], src_v, sem).wait()
    pltpu.async_copy(ei_hbm.at[pl.ds(E + base, EW)], dst_v, sem).wait()

    zeros = jnp.zeros((16,), jnp.int32)
    cnts[pl.ds(0, 16)] = zeros
    cnts[pl.ds(16, 16)] = zeros

    @pl.loop(0, EW // 16)
    def _(i):
        d = dst_v[pl.ds(i * 16, 16)]
        o = lax.shift_right_logical(d * _OMUL, _OSHR)
        cnt, last = plsc.scan_count(o)
        plsc.addupdate_scatter(cnts, [o], cnt, mask=last)

    c0 = cnts[pl.ds(0, 16)]
    c1 = cnts[pl.ds(16, 16)]
    p0 = jnp.bitwise_and(c0 + 7, -8)
    p1 = jnp.bitwise_and(c1 + 7, -8)
    i0 = plsc.cumsum(p0)
    s0 = jnp.sum(p0)
    i1 = plsc.cumsum(p1) + s0
    e0 = i0 - p0
    e1 = i1 - p1
    offv[pl.ds(0, 16)] = e0
    offv[pl.ds(16, 16)] = e1
    total = jnp.sum(p1) + s0
    offv[pl.ds(32, 16)] = jnp.full((16,), total, jnp.int32)
    curs[pl.ds(0, 16)] = e0
    curs[pl.ds(16, 16)] = e1

    @pl.loop(0, REGW // 16)
    def _(i):
        ebuf[pl.ds(i * 16, 16)] = jnp.full((16,), SENT, jnp.int32)

    @pl.loop(0, EW // 16)
    def _(i):
        s = src_v[pl.ds(i * 16, 16)]
        d = dst_v[pl.ds(i * 16, 16)]
        o = lax.shift_right_logical(d * _OMUL, _OSHR)
        cnt, last = plsc.scan_count(o)
        bpos = plsc.load_gather(curs, [o])
        pos = bpos + cnt - 1
        packed = jnp.bitwise_or(lax.shift_left(s, 14), d)
        plsc.store_scatter(ebuf, [pos], packed)
        plsc.addupdate_scatter(curs, [o], cnt, mask=last)

    pltpu.async_copy(ebuf, edges_hbm.at[pl.ds(w * REGW, REGW)], sem).wait()
    pltpu.async_copy(offv.at[pl.ds(0, OFFW)],
                     offs_hbm.at[pl.ds(w * OFFW, OFFW)], sem).wait()


@jax.jit
def _route(edge_index):
    kern = pl.kernel(
        _route_body,
        out_type=(
            jax.ShapeDtypeStruct((NW * REGW,), jnp.int32),
            jax.ShapeDtypeStruct((NW * OFFW,), jnp.int32),
        ),
        mesh=_mesh,
        compiler_params=_sc_params,
        scratch_types=[
            pltpu.VMEM((EW,), jnp.int32),
            pltpu.VMEM((EW,), jnp.int32),
            pltpu.VMEM((REGW,), jnp.int32),
            pltpu.VMEM((32,), jnp.int32),
            pltpu.VMEM((32,), jnp.int32),
            pltpu.VMEM((48,), jnp.int32),
            pltpu.SemaphoreType.DMA,
        ],
    )
    return kern(edge_index.reshape(2 * E))


KMAX = 2560


def _sget(ref, i):
    return jnp.max(plsc.load_gather(ref, [jnp.full((16,), i, jnp.int32)]))


def _agg_body(feat_hbm, edges_hbm, offs_hbm, agg_hbm, acc, rows0, rows1,
              eb0, eb1, si0, si1, dl0, dl1, offs_v, srb, slim,
              se0, sg0, sco):
    b = _wid()
    pltpu.async_copy(offs_hbm, offs_v, sco).wait()

    neg = jnp.full((16,), -jnp.inf, jnp.float32)

    @pl.loop(0, ROWPAD * 8)
    def _(r):
        acc[pl.ds(r * 16, 16)] = neg

    blo = b * P
    iota = lax.iota(jnp.int32, 16)

    def wbody(w, cursor):
        start = _sget(offs_v, w * OFFW + b)
        end = _sget(offs_v, w * OFFW + b + 1)
        seglen = end - start
        nch = lax.shift_right_logical(seglen + (CHUNK - 1), 7)
        ngr = lax.shift_right_logical(nch + 15, 4)
        segbase = w * REGW + start

        def gbody(g, z):
            k = g * 16 + iota
            m = k < nch
            off = k * CHUNK
            at = jnp.full((16,), cursor, jnp.int32) + k
            plsc.store_scatter(srb, [at],
                               jnp.full((16,), segbase, jnp.int32) + off,
                               mask=m)
            plsc.store_scatter(slim, [at],
                               jnp.full((16,), seglen, jnp.int32) - off,
                               mask=m)
            return z

        lax.fori_loop(0, ngr, gbody, 0)
        return cursor + nch

    K = lax.fori_loop(0, NW, wbody, 0)

    ebufs = (eb0, eb1)
    sidxs = (si0, si1)
    dlvs = (dl0, dl1)
    rowss = (rows0, rows1)
    sems_e = (se0.at[0], se0.at[1])
    sems_g = (sg0.at[0], sg0.at[1])

    def issue_edges(c, slot):
        @pl.when(c < K)
        def _():
            rb = pl.multiple_of(_sget(srb, c), 8)
            pltpu.async_copy(edges_hbm.at[pl.ds(rb, CHUNK)], ebufs[slot],
                             sems_e[slot])

    def stage_e(c, slot):
        @pl.when(c < K)
        def _():
            rb = pl.multiple_of(_sget(srb, c), 8)
            pltpu.async_copy(edges_hbm.at[pl.ds(rb, CHUNK)], ebufs[slot],
                             sems_e[slot]).wait()
            lim = _sget(slim, c)
            for j in range(CHUNK // 16):
                e = ebufs[slot][pl.ds(16 * j, 16)]
                sidxs[slot][pl.ds(16 * j, 16)] = lax.shift_right_logical(e, 14)
                dl = jnp.bitwise_and(e, SENT) - blo
                pos = iota + (16 * j)
                ok = jnp.logical_and(pos < lim,
                                     jnp.logical_and(dl >= 0, dl < P))
                dlvs[slot][pl.ds(16 * j, 16)] = jnp.where(ok, dl, TRASH)

    def fire_g(c, slot):
        @pl.when(c < K)
        def _():
            pltpu.async_copy(feat_hbm.at[sidxs[slot]],
                             rowss[slot].at[pl.ds(0, CHUNK)], sems_g[slot])

    def drain_g(c, slot):
        @pl.when(c < K)
        def _():
            pltpu.make_async_copy(feat_hbm.at[sidxs[slot]],
                                  rowss[slot].at[pl.ds(0, CHUNK)],
                                  sems_g[slot]).wait()

    def hotloop(c, slot):
        @pl.when(c < K)
        def _():
            @pl.loop(0, CHUNK)
            def _(j):
                dlb = plsc.load_gather(dlvs[slot],
                                       [jnp.full((16,), j, jnp.int32)])
                abase = dlb * 128 + iota
                for v in range(8):
                    addr = abase + (16 * v)
                    a = plsc.load_gather(acc, [addr])
                    r = rowss[slot][j, pl.ds(16 * v, 16)]
                    plsc.store_scatter(acc, [addr], jnp.maximum(a, r))

    def ibody(i, z):
        c0 = i * 2
        stage_e(c0, 0)
        stage_e(c0 + 1, 1)
        fire_g(c0, 0)
        fire_g(c0 + 1, 1)
        drain_g(c0, 0)
        drain_g(c0 + 1, 1)
        hotloop(c0, 0)
        hotloop(c0 + 1, 1)
        return z

    lax.fori_loop(0, lax.shift_right_logical(K + 1, 1), ibody, 0)

    pltpu.async_copy(acc.at[pl.ds(0, P * D)],
                     agg_hbm.at[pl.ds(blo * D, P * D)], sco).wait()


@jax.jit
def _aggregate(feat, edges, offs):
    kern = pl.kernel(
        _agg_body,
        out_type=jax.ShapeDtypeStruct((NPAD * D,), jnp.float32),
        mesh=_mesh,
        compiler_params=_sc_params,
        scratch_types=[
            pltpu.VMEM((ROWPAD * D,), jnp.float32),
            pltpu.VMEM((2 * CHUNK, D), jnp.float32),
            pltpu.VMEM((2 * CHUNK, D), jnp.float32),
            pltpu.VMEM((CHUNK,), jnp.int32),
            pltpu.VMEM((CHUNK,), jnp.int32),
            pltpu.VMEM((CHUNK,), jnp.int32),
            pltpu.VMEM((CHUNK,), jnp.int32),
            pltpu.VMEM((CHUNK,), jnp.int32),
            pltpu.VMEM((CHUNK,), jnp.int32),
            pltpu.VMEM((NW * OFFW,), jnp.int32),
            pltpu.VMEM((KMAX,), jnp.int32),
            pltpu.VMEM((KMAX,), jnp.int32),
            pltpu.SemaphoreType.DMA((2,)),
            pltpu.SemaphoreType.DMA((2,)),
            pltpu.SemaphoreType.DMA,
        ],
    )
    return kern(feat, edges, offs).reshape(NPAD, D)


_ROWS = 1000


def _dense_body(a_ref, x_ref, wlT_ref, bl_ref, wrT_ref, o_ref):
    a = a_ref[...]
    a = jnp.where(jnp.isneginf(a), 0.0, a)
    acc = jnp.dot(a, wlT_ref[...], preferred_element_type=jnp.float32)
    acc += jnp.dot(x_ref[...], wrT_ref[...], preferred_element_type=jnp.float32)
    acc += bl_ref[...]
    o_ref[...] = jnp.where(acc >= 0, acc, 0.01 * acc)


def _dense(agg, x, Wl, bl, Wr):
    grid = (N // _ROWS,)
    return pl.pallas_call(
        _dense_body,
        grid=grid,
        in_specs=[
            pl.BlockSpec((_ROWS, D), lambda i: (i, 0)),
            pl.BlockSpec((_ROWS, D), lambda i: (i, 0)),
            pl.BlockSpec((D, D), lambda i: (0, 0)),
            pl.BlockSpec((1, D), lambda i: (0, 0)),
            pl.BlockSpec((D, D), lambda i: (0, 0)),
        ],
        out_specs=pl.BlockSpec((_ROWS, D), lambda i: (i, 0)),
        out_shape=jax.ShapeDtypeStruct((N, D), jnp.float32),
    )(agg, x, Wl.T, bl.reshape(1, D), Wr.T)


def kernel(x, edge_index, Wl1, bl1, Wr1, Wl2, bl2, Wr2):
    edges, offs = _route(edge_index)
    agg1 = _aggregate(x, edges, offs)
    h = _dense(agg1, x, Wl1, bl1, Wr1)
    agg2 = _aggregate(h, edges, offs)
    return _dense(agg2, h, Wl2, bl2, Wr2)


# speedup vs baseline: 2.4973x; 1.0558x over previous
import dataclasses

import jax
import jax.numpy as jnp
from jax import lax
from jax.experimental import pallas as pl
from jax.experimental.pallas import tpu as pltpu
from jax.experimental.pallas import tpu_sc as plsc

N = 10000
E = 320000
D = 128

NC = 2
NS = 16
NW = NC * NS
EW = E // NW
P = 320
NPAD = NW * P
ROWPAD = P + 8
TRASH = P + 7
REGW = EW + 752
OFFW = 40
SENT = (1 << 14) - 1
CHUNK = 128

_OMUL = 6554
_OSHR = 21

_mesh = plsc.VectorSubcoreMesh(core_axis_name="c", subcore_axis_name="s")

_sc_params = pltpu.CompilerParams()
if "needs_layout_passes" in pltpu.CompilerParams.__dataclass_fields__:
    _sc_params = dataclasses.replace(_sc_params, needs_layout_passes=False)


def _wid():
    return lax.axis_index("s") * NC + lax.axis_index("c")


def _route_body(ei_hbm, edges_hbm, offs_hbm, src_v, dst_v, ebuf, cnts, curs,
                offv, sem):
    w = _wid()
    base = w * EW
    pltpu.async_copy(ei_hbm.at[pl.ds(base, EW)], src_v, sem).wait()
    pltpu.async_copy(ei_hbm.at[pl.ds(E + base, EW)], dst_v, sem).wait()

    zeros = jnp.zeros((16,), jnp.int32)
    cnts[pl.ds(0, 16)] = zeros
    cnts[pl.ds(16, 16)] = zeros

    @pl.loop(0, EW // 16)
    def _(i):
        d = dst_v[pl.ds(i * 16, 16)]
        o = lax.shift_right_logical(d * _OMUL, _OSHR)
        cnt, last = plsc.scan_count(o)
        plsc.addupdate_scatter(cnts, [o], cnt, mask=last)

    c0 = cnts[pl.ds(0, 16)]
    c1 = cnts[pl.ds(16, 16)]
    p0 = jnp.bitwise_and(c0 + 7, -8)
    p1 = jnp.bitwise_and(c1 + 7, -8)
    i0 = plsc.cumsum(p0)
    s0 = jnp.sum(p0)
    i1 = plsc.cumsum(p1) + s0
    e0 = i0 - p0
    e1 = i1 - p1
    offv[pl.ds(0, 16)] = e0
    offv[pl.ds(16, 16)] = e1
    total = jnp.sum(p1) + s0
    offv[pl.ds(32, 16)] = jnp.full((16,), total, jnp.int32)
    curs[pl.ds(0, 16)] = e0
    curs[pl.ds(16, 16)] = e1

    @pl.loop(0, REGW // 16)
    def _(i):
        ebuf[pl.ds(i * 16, 16)] = jnp.full((16,), SENT, jnp.int32)

    @pl.loop(0, EW // 16)
    def _(i):
        s = src_v[pl.ds(i * 16, 16)]
        d = dst_v[pl.ds(i * 16, 16)]
        o = lax.shift_right_logical(d * _OMUL, _OSHR)
        cnt, last = plsc.scan_count(o)
        bpos = plsc.load_gather(curs, [o])
        pos = bpos + cnt - 1
        packed = jnp.bitwise_or(lax.shift_left(s, 14), d)
        plsc.store_scatter(ebuf, [pos], packed)
        plsc.addupdate_scatter(curs, [o], cnt, mask=last)

    pltpu.async_copy(ebuf, edges_hbm.at[pl.ds(w * REGW, REGW)], sem).wait()
    pltpu.async_copy(offv.at[pl.ds(0, OFFW)],
                     offs_hbm.at[pl.ds(w * OFFW, OFFW)], sem).wait()


@jax.jit
def _route(edge_index):
    kern = pl.kernel(
        _route_body,
        out_type=(
            jax.ShapeDtypeStruct((NW * REGW,), jnp.int32),
            jax.ShapeDtypeStruct((NW * OFFW,), jnp.int32),
        ),
        mesh=_mesh,
        compiler_params=_sc_params,
        scratch_types=[
            pltpu.VMEM((EW,), jnp.int32),
            pltpu.VMEM((EW,), jnp.int32),
            pltpu.VMEM((REGW,), jnp.int32),
            pltpu.VMEM((32,), jnp.int32),
            pltpu.VMEM((32,), jnp.int32),
            pltpu.VMEM((48,), jnp.int32),
            pltpu.SemaphoreType.DMA,
        ],
    )
    return kern(edge_index.reshape(2 * E))


KMAX = 2560


def _sget(ref, i):
    return jnp.max(plsc.load_gather(ref, [jnp.full((16,), i, jnp.int32)]))


def _agg_body(feat_hbm, edges_hbm, offs_hbm, agg_hbm, acc, rowsb, ebb, sib,
              dlb_, offs_v, srb, slim, se0, sg0, sco):
    b = _wid()
    pltpu.async_copy(offs_hbm, offs_v, sco).wait()

    neg = jnp.full((16,), -jnp.inf, jnp.float32)

    @pl.loop(0, ROWPAD * 8)
    def _(r):
        acc[pl.ds(r * 16, 16)] = neg

    blo = b * P
    iota = lax.iota(jnp.int32, 16)

    def wbody(w, cursor):
        start = _sget(offs_v, w * OFFW + b)
        end = _sget(offs_v, w * OFFW + b + 1)
        seglen = end - start
        nch = lax.shift_right_logical(seglen + (CHUNK - 1), 7)
        ngr = lax.shift_right_logical(nch + 15, 4)
        segbase = w * REGW + start

        def gbody(g, z):
            k = g * 16 + iota
            m = k < nch
            off = k * CHUNK
            at = jnp.full((16,), cursor, jnp.int32) + k
            plsc.store_scatter(srb, [at],
                               jnp.full((16,), segbase, jnp.int32) + off,
                               mask=m)
            plsc.store_scatter(slim, [at],
                               jnp.full((16,), seglen, jnp.int32) - off,
                               mask=m)
            return z

        lax.fori_loop(0, ngr, gbody, 0)
        return cursor + nch

    K = lax.fori_loop(0, NW, wbody, 0)

    NB = 4
    ebufs = tuple(ebb.at[k] for k in range(NB))
    sidxs = tuple(sib.at[k] for k in range(NB))
    dlvs = tuple(dlb_.at[k] for k in range(NB))
    rowss = tuple(rowsb.at[k] for k in range(NB))
    sems_e = tuple(se0.at[k] for k in range(NB))
    sems_g = tuple(sg0.at[k] for k in range(NB))

    def issue_edges(c, slot):
        @pl.when(c < K)
        def _():
            rb = pl.multiple_of(_sget(srb, c), 8)
            pltpu.async_copy(edges_hbm.at[pl.ds(rb, CHUNK)], ebufs[slot],
                             sems_e[slot])

    def fire_e(c, slot):
        @pl.when(c < K)
        def _():
            rb = pl.multiple_of(_sget(srb, c), 8)
            pltpu.async_copy(edges_hbm.at[pl.ds(rb, CHUNK)], ebufs[slot],
                             sems_e[slot])

    def decode(c, slot):
        @pl.when(c < K)
        def _():
            pltpu.make_async_copy(edges_hbm.at[pl.ds(0, CHUNK)], ebufs[slot],
                                  sems_e[slot]).wait()
            lim = _sget(slim, c)
            for j in range(CHUNK // 16):
                e = ebufs[slot][pl.ds(16 * j, 16)]
                sidxs[slot][pl.ds(16 * j, 16)] = lax.shift_right_logical(e, 14)
                dl = jnp.bitwise_and(e, SENT) - blo
                pos = iota + (16 * j)
                ok = jnp.logical_and(pos < lim,
                                     jnp.logical_and(dl >= 0, dl < P))
                dlvs[slot][pl.ds(16 * j, 16)] = jnp.where(ok, dl, TRASH)

    def fire_g(c, slot):
        @pl.when(c < K)
        def _():
            pltpu.async_copy(feat_hbm.at[sidxs[slot]],
                             rowss[slot], sems_g[slot])

    def drain_g(c, slot):
        @pl.when(c < K)
        def _():
            pltpu.make_async_copy(feat_hbm.at[sidxs[slot]],
                                  rowss[slot], sems_g[slot]).wait()

    def hotloop(c, slot):
        @pl.when(c < K)
        def _():
            @pl.loop(0, CHUNK)
            def _(j):
                dlb = plsc.load_gather(dlvs[slot],
                                       [jnp.full((16,), j, jnp.int32)])
                abase = dlb * 128 + iota
                for v in range(8):
                    addr = abase + (16 * v)
                    a = plsc.load_gather(acc, [addr])
                    r = rowss[slot][j, pl.ds(16 * v, 16)]
                    plsc.store_scatter(acc, [addr], jnp.maximum(a, r))

    def ibody(i, z):
        c0 = i * NB
        for k in range(NB):
            fire_e(c0 + k, k)
        for k in range(NB):
            decode(c0 + k, k)
        for k in range(NB):
            fire_g(c0 + k, k)
        for k in range(NB):
            drain_g(c0 + k, k)
        for k in range(NB):
            hotloop(c0 + k, k)
        return z

    lax.fori_loop(0, lax.div(K + (NB - 1), NB), ibody, 0)

    pltpu.async_copy(acc.at[pl.ds(0, P * D)],
                     agg_hbm.at[pl.ds(blo * D, P * D)], sco).wait()


@jax.jit
def _aggregate(feat, edges, offs):
    kern = pl.kernel(
        _agg_body,
        out_type=jax.ShapeDtypeStruct((NPAD * D,), jnp.float32),
        mesh=_mesh,
        compiler_params=_sc_params,
        scratch_types=[
            pltpu.VMEM((ROWPAD * D,), jnp.float32),
            pltpu.VMEM((4, CHUNK, D), jnp.float32),
            pltpu.VMEM((4, CHUNK), jnp.int32),
            pltpu.VMEM((4, CHUNK), jnp.int32),
            pltpu.VMEM((4, CHUNK), jnp.int32),
            pltpu.VMEM((NW * OFFW,), jnp.int32),
            pltpu.VMEM((KMAX,), jnp.int32),
            pltpu.VMEM((KMAX,), jnp.int32),
            pltpu.SemaphoreType.DMA((4,)),
            pltpu.SemaphoreType.DMA((4,)),
            pltpu.SemaphoreType.DMA,
        ],
    )
    return kern(feat, edges, offs).reshape(NPAD, D)


_ROWS = 1000


def _dense_body(a_ref, x_ref, wlT_ref, bl_ref, wrT_ref, o_ref):
    a = a_ref[...]
    a = jnp.where(jnp.isneginf(a), 0.0, a)
    acc = jnp.dot(a, wlT_ref[...], preferred_element_type=jnp.float32)
    acc += jnp.dot(x_ref[...], wrT_ref[...], preferred_element_type=jnp.float32)
    acc += bl_ref[...]
    o_ref[...] = jnp.where(acc >= 0, acc, 0.01 * acc)


def _dense(agg, x, Wl, bl, Wr):
    grid = (N // _ROWS,)
    return pl.pallas_call(
        _dense_body,
        grid=grid,
        in_specs=[
            pl.BlockSpec((_ROWS, D), lambda i: (i, 0)),
            pl.BlockSpec((_ROWS, D), lambda i: (i, 0)),
            pl.BlockSpec((D, D), lambda i: (0, 0)),
            pl.BlockSpec((1, D), lambda i: (0, 0)),
            pl.BlockSpec((D, D), lambda i: (0, 0)),
        ],
        out_specs=pl.BlockSpec((_ROWS, D), lambda i: (i, 0)),
        out_shape=jax.ShapeDtypeStruct((N, D), jnp.float32),
    )(agg, x, Wl.T, bl.reshape(1, D), Wr.T)


def kernel(x, edge_index, Wl1, bl1, Wr1, Wl2, bl2, Wr2):
    edges, offs = _route(edge_index)
    agg1 = _aggregate(x, edges, offs)
    h = _dense(agg1, x, Wl1, bl1, Wr1)
    agg2 = _aggregate(h, edges, offs)
    return _dense(agg2, h, Wl2, bl2, Wr2)


# speedup vs baseline: 4.1011x; 1.6422x over previous
import dataclasses

import jax
import jax.numpy as jnp
from jax import lax
from jax.experimental import pallas as pl
from jax.experimental.pallas import tpu as pltpu
from jax.experimental.pallas import tpu_sc as plsc

N = 10000
E = 320000
D = 128

NC = 2
NS = 16
NW = NC * NS
EW = E // NW
P = 320
NPAD = NW * P
ROWPAD = P + 8
TRASH = P + 7
REGW = EW + 752
OFFW = 40
SENT = (1 << 14) - 1
CHUNK = 128

_OMUL = 6554
_OSHR = 21

_mesh = plsc.VectorSubcoreMesh(core_axis_name="c", subcore_axis_name="s")

_sc_params = pltpu.CompilerParams()
if "needs_layout_passes" in pltpu.CompilerParams.__dataclass_fields__:
    _sc_params = dataclasses.replace(_sc_params, needs_layout_passes=False)


def _wid():
    return lax.axis_index("s") * NC + lax.axis_index("c")


def _route_body(ei_hbm, edges_hbm, offs_hbm, src_v, dst_v, ebuf, cnts, curs,
                offv, sem):
    w = _wid()
    base = w * EW
    pltpu.async_copy(ei_hbm.at[pl.ds(base, EW)], src_v, sem).wait()
    pltpu.async_copy(ei_hbm.at[pl.ds(E + base, EW)], dst_v, sem).wait()

    zeros = jnp.zeros((16,), jnp.int32)
    cnts[pl.ds(0, 16)] = zeros
    cnts[pl.ds(16, 16)] = zeros

    @pl.loop(0, EW // 16)
    def _(i):
        d = dst_v[pl.ds(i * 16, 16)]
        o = lax.shift_right_logical(d * _OMUL, _OSHR)
        cnt, last = plsc.scan_count(o)
        plsc.addupdate_scatter(cnts, [o], cnt, mask=last)

    c0 = cnts[pl.ds(0, 16)]
    c1 = cnts[pl.ds(16, 16)]
    p0 = jnp.bitwise_and(c0 + 7, -8)
    p1 = jnp.bitwise_and(c1 + 7, -8)
    i0 = plsc.cumsum(p0)
    s0 = jnp.sum(p0)
    i1 = plsc.cumsum(p1) + s0
    e0 = i0 - p0
    e1 = i1 - p1
    offv[pl.ds(0, 16)] = e0
    offv[pl.ds(16, 16)] = e1
    total = jnp.sum(p1) + s0
    offv[pl.ds(32, 16)] = jnp.full((16,), total, jnp.int32)
    curs[pl.ds(0, 16)] = e0
    curs[pl.ds(16, 16)] = e1

    @pl.loop(0, REGW // 16)
    def _(i):
        ebuf[pl.ds(i * 16, 16)] = jnp.full((16,), SENT, jnp.int32)

    @pl.loop(0, EW // 16)
    def _(i):
        s = src_v[pl.ds(i * 16, 16)]
        d = dst_v[pl.ds(i * 16, 16)]
        o = lax.shift_right_logical(d * _OMUL, _OSHR)
        cnt, last = plsc.scan_count(o)
        bpos = plsc.load_gather(curs, [o])
        pos = bpos + cnt - 1
        packed = jnp.bitwise_or(lax.shift_left(s, 14), d)
        plsc.store_scatter(ebuf, [pos], packed)
        plsc.addupdate_scatter(curs, [o], cnt, mask=last)

    pltpu.async_copy(ebuf, edges_hbm.at[pl.ds(w * REGW, REGW)], sem).wait()
    pltpu.async_copy(offv.at[pl.ds(0, OFFW)],
                     offs_hbm.at[pl.ds(w * OFFW, OFFW)], sem).wait()


@jax.jit
def _route(edge_index):
    kern = pl.kernel(
        _route_body,
        out_type=(
            jax.ShapeDtypeStruct((NW * REGW,), jnp.int32),
            jax.ShapeDtypeStruct((NW * OFFW,), jnp.int32),
        ),
        mesh=_mesh,
        compiler_params=_sc_params,
        scratch_types=[
            pltpu.VMEM((EW,), jnp.int32),
            pltpu.VMEM((EW,), jnp.int32),
            pltpu.VMEM((REGW,), jnp.int32),
            pltpu.VMEM((32,), jnp.int32),
            pltpu.VMEM((32,), jnp.int32),
            pltpu.VMEM((48,), jnp.int32),
            pltpu.SemaphoreType.DMA,
        ],
    )
    return kern(edge_index.reshape(2 * E))


KMAX = 2560


def _sget(ref, i):
    return jnp.max(plsc.load_gather(ref, [jnp.full((16,), i, jnp.int32)]))


def _agg_body(feat_hbm, edges_hbm, offs_hbm, agg_hbm, acc, rowsb, ebb, sib,
              dlb_, offs_v, srb, slim, se0, sg0, sco):
    b = _wid()
    pltpu.async_copy(offs_hbm, offs_v, sco).wait()

    neg = jnp.full((16,), -jnp.inf, jnp.float32)

    @pl.loop(0, ROWPAD * 8)
    def _(r):
        acc[pl.ds(r * 16, 16)] = neg

    blo = b * P
    iota = lax.iota(jnp.int32, 16)

    def wbody(w, cursor):
        start = _sget(offs_v, w * OFFW + b)
        end = _sget(offs_v, w * OFFW + b + 1)
        seglen = end - start
        nch = lax.shift_right_logical(seglen + (CHUNK - 1), 7)
        ngr = lax.shift_right_logical(nch + 15, 4)
        segbase = w * REGW + start

        def gbody(g, z):
            k = g * 16 + iota
            m = k < nch
            off = k * CHUNK
            at = jnp.full((16,), cursor, jnp.int32) + k
            plsc.store_scatter(srb, [at],
                               jnp.full((16,), segbase, jnp.int32) + off,
                               mask=m)
            plsc.store_scatter(slim, [at],
                               jnp.full((16,), seglen, jnp.int32) - off,
                               mask=m)
            return z

        lax.fori_loop(0, ngr, gbody, 0)
        return cursor + nch

    K = lax.fori_loop(0, NW, wbody, 0)

    NB = 4
    ebufs = tuple(ebb.at[k] for k in range(NB))
    sidxs = tuple(sib.at[k] for k in range(NB))
    dlvs = tuple(dlb_.at[k] for k in range(NB))
    rowss = tuple(rowsb.at[k] for k in range(NB))
    sems_e = tuple(se0.at[k] for k in range(NB))
    sems_g = tuple(sg0.at[k] for k in range(NB))

    def issue_edges(c, slot):
        @pl.when(c < K)
        def _():
            rb = pl.multiple_of(_sget(srb, c), 8)
            pltpu.async_copy(edges_hbm.at[pl.ds(rb, CHUNK)], ebufs[slot],
                             sems_e[slot])

    def fire_e(c, slot):
        @pl.when(c < K)
        def _():
            rb = pl.multiple_of(_sget(srb, c), 8)
            pltpu.async_copy(edges_hbm.at[pl.ds(rb, CHUNK)], ebufs[slot],
                             sems_e[slot])

    def decode(c, slot):
        @pl.when(c < K)
        def _():
            pltpu.make_async_copy(edges_hbm.at[pl.ds(0, CHUNK)], ebufs[slot],
                                  sems_e[slot]).wait()
            lim = _sget(slim, c)
            for j in range(CHUNK // 16):
                e = ebufs[slot][pl.ds(16 * j, 16)]
                sidxs[slot][pl.ds(16 * j, 16)] = lax.shift_right_logical(e, 14)
                dl = jnp.bitwise_and(e, SENT) - blo
                pos = iota + (16 * j)
                ok = jnp.logical_and(pos < lim,
                                     jnp.logical_and(dl >= 0, dl < P))
                dlvs[slot][pl.ds(16 * j, 16)] = jnp.where(ok, dl, TRASH) * 128

    def fire_g(c, slot):
        @pl.when(c < K)
        def _():
            pltpu.async_copy(feat_hbm.at[sidxs[slot]],
                             rowss[slot], sems_g[slot])

    def drain_g(c, slot):
        @pl.when(c < K)
        def _():
            pltpu.make_async_copy(feat_hbm.at[sidxs[slot]],
                                  rowss[slot], sems_g[slot]).wait()

    def one_edge(slot, j):
        dlb = plsc.load_gather(dlvs[slot], [jnp.full((16,), j, jnp.int32)])
        abase = dlb + iota
        addrs = [abase + (16 * v) for v in range(8)]
        accs = [plsc.load_gather(acc, [a]) for a in addrs]
        rws = [rowss[slot][j, pl.ds(16 * v, 16)] for v in range(8)]
        mx = [jnp.maximum(a, r) for a, r in zip(accs, rws)]
        for a, m in zip(addrs, mx):
            plsc.store_scatter(acc, [a], m)

    def hotloop(c, slot):
        @pl.when(c < K)
        def _():
            @pl.loop(0, CHUNK, step=2)
            def _(j):
                one_edge(slot, j)
                one_edge(slot, j + 1)

    def ibody(i, z):
        c0 = i * NB
        for k in range(NB):
            fire_e(c0 + k, k)
        for k in range(NB):
            decode(c0 + k, k)
        for k in range(NB):
            fire_g(c0 + k, k)
        for k in range(NB):
            drain_g(c0 + k, k)
        for k in range(NB):
            hotloop(c0 + k, k)
        return z

    lax.fori_loop(0, lax.div(K + (NB - 1), NB), ibody, 0)

    pltpu.async_copy(acc.at[pl.ds(0, P * D)],
                     agg_hbm.at[pl.ds(blo * D, P * D)], sco).wait()


@jax.jit
def _aggregate(feat, edges, offs):
    kern = pl.kernel(
        _agg_body,
        out_type=jax.ShapeDtypeStruct((NPAD * D,), jnp.float32),
        mesh=_mesh,
        compiler_params=_sc_params,
        scratch_types=[
            pltpu.VMEM((ROWPAD * D,), jnp.float32),
            pltpu.VMEM((4, CHUNK, D), jnp.float32),
            pltpu.VMEM((4, CHUNK), jnp.int32),
            pltpu.VMEM((4, CHUNK), jnp.int32),
            pltpu.VMEM((4, CHUNK), jnp.int32),
            pltpu.VMEM((NW * OFFW,), jnp.int32),
            pltpu.VMEM((KMAX,), jnp.int32),
            pltpu.VMEM((KMAX,), jnp.int32),
            pltpu.SemaphoreType.DMA((4,)),
            pltpu.SemaphoreType.DMA((4,)),
            pltpu.SemaphoreType.DMA,
        ],
    )
    return kern(feat, edges, offs).reshape(NPAD, D)


_ROWS = 1000


def _dense_body(a_ref, x_ref, wlT_ref, bl_ref, wrT_ref, o_ref):
    a = a_ref[...]
    a = jnp.where(jnp.isneginf(a), 0.0, a)
    acc = jnp.dot(a, wlT_ref[...], preferred_element_type=jnp.float32)
    acc += jnp.dot(x_ref[...], wrT_ref[...], preferred_element_type=jnp.float32)
    acc += bl_ref[...]
    o_ref[...] = jnp.where(acc >= 0, acc, 0.01 * acc)


def _dense(agg, x, Wl, bl, Wr):
    grid = (N // _ROWS,)
    return pl.pallas_call(
        _dense_body,
        grid=grid,
        in_specs=[
            pl.BlockSpec((_ROWS, D), lambda i: (i, 0)),
            pl.BlockSpec((_ROWS, D), lambda i: (i, 0)),
            pl.BlockSpec((D, D), lambda i: (0, 0)),
            pl.BlockSpec((1, D), lambda i: (0, 0)),
            pl.BlockSpec((D, D), lambda i: (0, 0)),
        ],
        out_specs=pl.BlockSpec((_ROWS, D), lambda i: (i, 0)),
        out_shape=jax.ShapeDtypeStruct((N, D), jnp.float32),
    )(agg, x, Wl.T, bl.reshape(1, D), Wr.T)


def kernel(x, edge_index, Wl1, bl1, Wr1, Wl2, bl2, Wr2):
    edges, offs = _route(edge_index)
    agg1 = _aggregate(x, edges, offs)
    h = _dense(agg1, x, Wl1, bl1, Wr1)
    agg2 = _aggregate(h, edges, offs)
    return _dense(agg2, h, Wl2, bl2, Wr2)


# speedup vs baseline: 4.1367x; 1.0087x over previous
import dataclasses

import jax
import jax.numpy as jnp
from jax import lax
from jax.experimental import pallas as pl
from jax.experimental.pallas import tpu as pltpu
from jax.experimental.pallas import tpu_sc as plsc

N = 10000
E = 320000
D = 128

NC = 2
NS = 16
NW = NC * NS
EW = E // NW
P = 320
NPAD = NW * P
ROWPAD = P + 8
TRASH = P + 7
REGW = EW + 752
OFFW = 40
SENT = (1 << 14) - 1
CHUNK = 128

_OMUL = 6554
_OSHR = 21

_mesh = plsc.VectorSubcoreMesh(core_axis_name="c", subcore_axis_name="s")

_sc_params = pltpu.CompilerParams()
if "needs_layout_passes" in pltpu.CompilerParams.__dataclass_fields__:
    _sc_params = dataclasses.replace(_sc_params, needs_layout_passes=False)


def _wid():
    return lax.axis_index("s") * NC + lax.axis_index("c")


def _route_body(ei_hbm, edges_hbm, offs_hbm, src_v, dst_v, ebuf, cnts, curs,
                offv, sem):
    w = _wid()
    base = w * EW
    pltpu.async_copy(ei_hbm.at[pl.ds(base, EW)], src_v, sem).wait()
    pltpu.async_copy(ei_hbm.at[pl.ds(E + base, EW)], dst_v, sem).wait()

    zeros = jnp.zeros((16,), jnp.int32)
    cnts[pl.ds(0, 16)] = zeros
    cnts[pl.ds(16, 16)] = zeros

    @pl.loop(0, EW // 16)
    def _(i):
        d = dst_v[pl.ds(i * 16, 16)]
        o = lax.shift_right_logical(d * _OMUL, _OSHR)
        cnt, last = plsc.scan_count(o)
        plsc.addupdate_scatter(cnts, [o], cnt, mask=last)

    c0 = cnts[pl.ds(0, 16)]
    c1 = cnts[pl.ds(16, 16)]
    p0 = jnp.bitwise_and(c0 + 7, -8)
    p1 = jnp.bitwise_and(c1 + 7, -8)
    i0 = plsc.cumsum(p0)
    s0 = jnp.sum(p0)
    i1 = plsc.cumsum(p1) + s0
    e0 = i0 - p0
    e1 = i1 - p1
    offv[pl.ds(0, 16)] = e0
    offv[pl.ds(16, 16)] = e1
    total = jnp.sum(p1) + s0
    offv[pl.ds(32, 16)] = jnp.full((16,), total, jnp.int32)
    curs[pl.ds(0, 16)] = e0
    curs[pl.ds(16, 16)] = e1

    @pl.loop(0, REGW // 16)
    def _(i):
        ebuf[pl.ds(i * 16, 16)] = jnp.full((16,), SENT, jnp.int32)

    @pl.loop(0, EW // 16)
    def _(i):
        s = src_v[pl.ds(i * 16, 16)]
        d = dst_v[pl.ds(i * 16, 16)]
        o = lax.shift_right_logical(d * _OMUL, _OSHR)
        cnt, last = plsc.scan_count(o)
        bpos = plsc.load_gather(curs, [o])
        pos = bpos + cnt - 1
        packed = jnp.bitwise_or(lax.shift_left(s, 14), d)
        plsc.store_scatter(ebuf, [pos], packed)
        plsc.addupdate_scatter(curs, [o], cnt, mask=last)

    pltpu.async_copy(ebuf, edges_hbm.at[pl.ds(w * REGW, REGW)], sem).wait()
    pltpu.async_copy(offv.at[pl.ds(0, OFFW)],
                     offs_hbm.at[pl.ds(w * OFFW, OFFW)], sem).wait()


@jax.jit
def _route(edge_index):
    kern = pl.kernel(
        _route_body,
        out_type=(
            jax.ShapeDtypeStruct((NW * REGW,), jnp.int32),
            jax.ShapeDtypeStruct((NW * OFFW,), jnp.int32),
        ),
        mesh=_mesh,
        compiler_params=_sc_params,
        scratch_types=[
            pltpu.VMEM((EW,), jnp.int32),
            pltpu.VMEM((EW,), jnp.int32),
            pltpu.VMEM((REGW,), jnp.int32),
            pltpu.VMEM((32,), jnp.int32),
            pltpu.VMEM((32,), jnp.int32),
            pltpu.VMEM((48,), jnp.int32),
            pltpu.SemaphoreType.DMA,
        ],
    )
    return kern(edge_index.reshape(2 * E))


KMAX = 2560


def _sget(ref, i):
    return jnp.max(plsc.load_gather(ref, [jnp.full((16,), i, jnp.int32)]))


def _agg_body(feat_hbm, edges_hbm, offs_hbm, agg_hbm, acc, rowsb, ebb, sib,
              dlb_, offs_v, srb, slim, se0, sg0, sco):
    b = _wid()
    pltpu.async_copy(offs_hbm, offs_v, sco).wait()

    neg = jnp.full((16,), -jnp.inf, jnp.float32)

    @pl.loop(0, ROWPAD * 8)
    def _(r):
        acc[pl.ds(r * 16, 16)] = neg

    blo = b * P
    iota = lax.iota(jnp.int32, 16)

    def wbody(w, cursor):
        start = _sget(offs_v, w * OFFW + b)
        end = _sget(offs_v, w * OFFW + b + 1)
        seglen = end - start
        nch = lax.shift_right_logical(seglen + (CHUNK - 1), 7)
        ngr = lax.shift_right_logical(nch + 15, 4)
        segbase = w * REGW + start

        def gbody(g, z):
            k = g * 16 + iota
            m = k < nch
            off = k * CHUNK
            at = jnp.full((16,), cursor, jnp.int32) + k
            plsc.store_scatter(srb, [at],
                               jnp.full((16,), segbase, jnp.int32) + off,
                               mask=m)
            plsc.store_scatter(slim, [at],
                               jnp.full((16,), seglen, jnp.int32) - off,
                               mask=m)
            return z

        lax.fori_loop(0, ngr, gbody, 0)
        return cursor + nch

    K = lax.fori_loop(0, NW, wbody, 0)

    NB = 4
    ebufs = tuple(ebb.at[k] for k in range(NB))
    sidxs = tuple(sib.at[k] for k in range(NB))
    dlvs = tuple(dlb_.at[k] for k in range(NB))
    rowss = tuple(rowsb.at[k] for k in range(NB))
    sems_e = tuple(se0.at[k] for k in range(NB))
    sems_g = tuple(sg0.at[k] for k in range(NB))

    def issue_edges(c, slot):
        @pl.when(c < K)
        def _():
            rb = pl.multiple_of(_sget(srb, c), 8)
            pltpu.async_copy(edges_hbm.at[pl.ds(rb, CHUNK)], ebufs[slot],
                             sems_e[slot])

    def fire_e(c, slot):
        @pl.when(c < K)
        def _():
            rb = pl.multiple_of(_sget(srb, c), 8)
            pltpu.async_copy(edges_hbm.at[pl.ds(rb, CHUNK)], ebufs[slot],
                             sems_e[slot])

    def decode(c, slot):
        @pl.when(c < K)
        def _():
            pltpu.make_async_copy(edges_hbm.at[pl.ds(0, CHUNK)], ebufs[slot],
                                  sems_e[slot]).wait()
            lim = _sget(slim, c)
            for j in range(CHUNK // 16):
                e = ebufs[slot][pl.ds(16 * j, 16)]
                sidxs[slot][pl.ds(16 * j, 16)] = lax.shift_right_logical(e, 14)
                dl = jnp.bitwise_and(e, SENT) - blo
                pos = iota + (16 * j)
                ok = jnp.logical_and(pos < lim,
                                     jnp.logical_and(dl >= 0, dl < P))
                dlvs[slot][pl.ds(16 * j, 16)] = jnp.where(ok, dl, TRASH) * 128

    def fire_g(c, slot):
        @pl.when(c < K)
        def _():
            pltpu.async_copy(feat_hbm.at[sidxs[slot]],
                             rowss[slot], sems_g[slot])

    def drain_g(c, slot):
        @pl.when(c < K)
        def _():
            pltpu.make_async_copy(feat_hbm.at[sidxs[slot]],
                                  rowss[slot], sems_g[slot]).wait()

    def one_edge(slot, j):
        dlb = plsc.load_gather(dlvs[slot], [jnp.full((16,), j, jnp.int32)])
        abase = dlb + iota
        addrs = [abase + (16 * v) for v in range(8)]
        accs = [plsc.load_gather(acc, [a]) for a in addrs]
        rws = [rowss[slot][j, pl.ds(16 * v, 16)] for v in range(8)]
        mx = [jnp.maximum(a, r) for a, r in zip(accs, rws)]
        for a, m in zip(addrs, mx):
            plsc.store_scatter(acc, [a], m)

    def hotloop(c, slot):
        @pl.when(c < K)
        def _():
            @pl.loop(0, CHUNK, step=4)
            def _(j):
                one_edge(slot, j)
                one_edge(slot, j + 1)
                one_edge(slot, j + 2)
                one_edge(slot, j + 3)

    def ibody(i, z):
        c0 = i * NB
        for k in range(NB):
            fire_e(c0 + k, k)
        for k in range(NB):
            decode(c0 + k, k)
        for k in range(NB):
            fire_g(c0 + k, k)
        for k in range(NB):
            drain_g(c0 + k, k)
        for k in range(NB):
            hotloop(c0 + k, k)
        return z

    lax.fori_loop(0, lax.div(K + (NB - 1), NB), ibody, 0)

    pltpu.async_copy(acc.at[pl.ds(0, P * D)],
                     agg_hbm.at[pl.ds(blo * D, P * D)], sco).wait()


@jax.jit
def _aggregate(feat, edges, offs):
    kern = pl.kernel(
        _agg_body,
        out_type=jax.ShapeDtypeStruct((NPAD * D,), jnp.float32),
        mesh=_mesh,
        compiler_params=_sc_params,
        scratch_types=[
            pltpu.VMEM((ROWPAD * D,), jnp.float32),
            pltpu.VMEM((4, CHUNK, D), jnp.float32),
            pltpu.VMEM((4, CHUNK), jnp.int32),
            pltpu.VMEM((4, CHUNK), jnp.int32),
            pltpu.VMEM((4, CHUNK), jnp.int32),
            pltpu.VMEM((NW * OFFW,), jnp.int32),
            pltpu.VMEM((KMAX,), jnp.int32),
            pltpu.VMEM((KMAX,), jnp.int32),
            pltpu.SemaphoreType.DMA((4,)),
            pltpu.SemaphoreType.DMA((4,)),
            pltpu.SemaphoreType.DMA,
        ],
    )
    return kern(feat, edges, offs).reshape(NPAD, D)


_ROWS = 1000


def _dense_body(a_ref, x_ref, wlT_ref, bl_ref, wrT_ref, o_ref):
    a = a_ref[...]
    a = jnp.where(jnp.isneginf(a), 0.0, a)
    acc = jnp.dot(a, wlT_ref[...], preferred_element_type=jnp.float32)
    acc += jnp.dot(x_ref[...], wrT_ref[...], preferred_element_type=jnp.float32)
    acc += bl_ref[...]
    o_ref[...] = jnp.where(acc >= 0, acc, 0.01 * acc)


def _dense(agg, x, Wl, bl, Wr):
    grid = (N // _ROWS,)
    return pl.pallas_call(
        _dense_body,
        grid=grid,
        in_specs=[
            pl.BlockSpec((_ROWS, D), lambda i: (i, 0)),
            pl.BlockSpec((_ROWS, D), lambda i: (i, 0)),
            pl.BlockSpec((D, D), lambda i: (0, 0)),
            pl.BlockSpec((1, D), lambda i: (0, 0)),
            pl.BlockSpec((D, D), lambda i: (0, 0)),
        ],
        out_specs=pl.BlockSpec((_ROWS, D), lambda i: (i, 0)),
        out_shape=jax.ShapeDtypeStruct((N, D), jnp.float32),
    )(agg, x, Wl.T, bl.reshape(1, D), Wr.T)


def kernel(x, edge_index, Wl1, bl1, Wr1, Wl2, bl2, Wr2):
    edges, offs = _route(edge_index)
    agg1 = _aggregate(x, edges, offs)
    h = _dense(agg1, x, Wl1, bl1, Wr1)
    agg2 = _aggregate(h, edges, offs)
    return _dense(agg2, h, Wl2, bl2, Wr2)


# speedup vs baseline: 4.1369x; 1.0000x over previous
import dataclasses

import jax
import jax.numpy as jnp
from jax import lax
from jax.experimental import pallas as pl
from jax.experimental.pallas import tpu as pltpu
from jax.experimental.pallas import tpu_sc as plsc

N = 10000
E = 320000
D = 128

NC = 2
NS = 16
NW = NC * NS
EW = E // NW
P = 320
NPAD = NW * P
ROWPAD = P + 8
TRASH = P + 7
REGW = EW + 752
OFFW = 40
SENT = (1 << 14) - 1
CHUNK = 128

_OMUL = 6554
_OSHR = 21

_mesh = plsc.VectorSubcoreMesh(core_axis_name="c", subcore_axis_name="s")

_sc_params = pltpu.CompilerParams()
if "needs_layout_passes" in pltpu.CompilerParams.__dataclass_fields__:
    _sc_params = dataclasses.replace(_sc_params, needs_layout_passes=False)


def _wid():
    return lax.axis_index("s") * NC + lax.axis_index("c")


def _route_body(ei_hbm, edges_hbm, offs_hbm, src_v, dst_v, ebuf, cnts, curs,
                offv, sem):
    w = _wid()
    base = w * EW
    pltpu.async_copy(ei_hbm.at[pl.ds(base, EW)], src_v, sem).wait()
    pltpu.async_copy(ei_hbm.at[pl.ds(E + base, EW)], dst_v, sem).wait()

    zeros = jnp.zeros((16,), jnp.int32)
    cnts[pl.ds(0, 16)] = zeros
    cnts[pl.ds(16, 16)] = zeros

    @pl.loop(0, EW // 16)
    def _(i):
        d = dst_v[pl.ds(i * 16, 16)]
        o = lax.shift_right_logical(d * _OMUL, _OSHR)
        cnt, last = plsc.scan_count(o)
        plsc.addupdate_scatter(cnts, [o], cnt, mask=last)

    c0 = cnts[pl.ds(0, 16)]
    c1 = cnts[pl.ds(16, 16)]
    p0 = jnp.bitwise_and(c0 + 7, -8)
    p1 = jnp.bitwise_and(c1 + 7, -8)
    i0 = plsc.cumsum(p0)
    s0 = jnp.sum(p0)
    i1 = plsc.cumsum(p1) + s0
    e0 = i0 - p0
    e1 = i1 - p1
    offv[pl.ds(0, 16)] = e0
    offv[pl.ds(16, 16)] = e1
    total = jnp.sum(p1) + s0
    offv[pl.ds(32, 16)] = jnp.full((16,), total, jnp.int32)
    curs[pl.ds(0, 16)] = e0
    curs[pl.ds(16, 16)] = e1

    @pl.loop(0, REGW // 16)
    def _(i):
        ebuf[pl.ds(i * 16, 16)] = jnp.full((16,), SENT, jnp.int32)

    @pl.loop(0, EW // 16)
    def _(i):
        s = src_v[pl.ds(i * 16, 16)]
        d = dst_v[pl.ds(i * 16, 16)]
        o = lax.shift_right_logical(d * _OMUL, _OSHR)
        cnt, last = plsc.scan_count(o)
        bpos = plsc.load_gather(curs, [o])
        pos = bpos + cnt - 1
        packed = jnp.bitwise_or(lax.shift_left(s, 14), d)
        plsc.store_scatter(ebuf, [pos], packed)
        plsc.addupdate_scatter(curs, [o], cnt, mask=last)

    pltpu.async_copy(ebuf, edges_hbm.at[pl.ds(w * REGW, REGW)], sem).wait()
    pltpu.async_copy(offv.at[pl.ds(0, OFFW)],
                     offs_hbm.at[pl.ds(w * OFFW, OFFW)], sem).wait()


@jax.jit
def _route(edge_index):
    kern = pl.kernel(
        _route_body,
        out_type=(
            jax.ShapeDtypeStruct((NW * REGW,), jnp.int32),
            jax.ShapeDtypeStruct((NW * OFFW,), jnp.int32),
        ),
        mesh=_mesh,
        compiler_params=_sc_params,
        scratch_types=[
            pltpu.VMEM((EW,), jnp.int32),
            pltpu.VMEM((EW,), jnp.int32),
            pltpu.VMEM((REGW,), jnp.int32),
            pltpu.VMEM((32,), jnp.int32),
            pltpu.VMEM((32,), jnp.int32),
            pltpu.VMEM((48,), jnp.int32),
            pltpu.SemaphoreType.DMA,
        ],
    )
    return kern(edge_index.reshape(2 * E))


KMAX = 2560


def _sget(ref, i):
    return jnp.max(plsc.load_gather(ref, [jnp.full((16,), i, jnp.int32)]))


def _agg_body(feat_hbm, edges_hbm, offs_hbm, agg_hbm, acc, rowsb, ebb, sib,
              dlb_, offs_v, srb, slim, se0, sg0, sco):
    b = _wid()
    pltpu.async_copy(offs_hbm, offs_v, sco).wait()

    neg = jnp.full((16,), -jnp.inf, jnp.float32)

    @pl.loop(0, ROWPAD * 8)
    def _(r):
        acc[pl.ds(r * 16, 16)] = neg

    blo = b * P
    iota = lax.iota(jnp.int32, 16)

    def wbody(w, cursor):
        start = _sget(offs_v, w * OFFW + b)
        end = _sget(offs_v, w * OFFW + b + 1)
        seglen = end - start
        nch = lax.shift_right_logical(seglen + (CHUNK - 1), 7)
        ngr = lax.shift_right_logical(nch + 15, 4)
        segbase = w * REGW + start

        def gbody(g, z):
            k = g * 16 + iota
            m = k < nch
            off = k * CHUNK
            at = jnp.full((16,), cursor, jnp.int32) + k
            plsc.store_scatter(srb, [at],
                               jnp.full((16,), segbase, jnp.int32) + off,
                               mask=m)
            plsc.store_scatter(slim, [at],
                               jnp.full((16,), seglen, jnp.int32) - off,
                               mask=m)
            return z

        lax.fori_loop(0, ngr, gbody, 0)
        return cursor + nch

    K = lax.fori_loop(0, NW, wbody, 0)

    NB = 4
    ebufs = tuple(ebb.at[k] for k in range(NB))
    sidxs = tuple(sib.at[k] for k in range(NB))
    dlvs = tuple(dlb_.at[k] for k in range(NB))
    rowss = tuple(rowsb.at[k] for k in range(NB))
    sems_e = tuple(se0.at[k] for k in range(NB))
    sems_g = tuple(sg0.at[k] for k in range(NB))

    def issue_edges(c, slot):
        @pl.when(c < K)
        def _():
            rb = pl.multiple_of(_sget(srb, c), 8)
            pltpu.async_copy(edges_hbm.at[pl.ds(rb, CHUNK)], ebufs[slot],
                             sems_e[slot])

    def fire_e(c, slot):
        @pl.when(c < K)
        def _():
            rb = pl.multiple_of(_sget(srb, c), 8)
            pltpu.async_copy(edges_hbm.at[pl.ds(rb, CHUNK)], ebufs[slot],
                             sems_e[slot])

    def decode(c, slot):
        @pl.when(c < K)
        def _():
            pltpu.make_async_copy(edges_hbm.at[pl.ds(0, CHUNK)], ebufs[slot],
                                  sems_e[slot]).wait()
            lim = _sget(slim, c)
            for j in range(CHUNK // 16):
                e = ebufs[slot][pl.ds(16 * j, 16)]
                sidxs[slot][pl.ds(16 * j, 16)] = lax.shift_right_logical(e, 14)
                dl = jnp.bitwise_and(e, SENT) - blo
                pos = iota + (16 * j)
                ok = jnp.logical_and(pos < lim,
                                     jnp.logical_and(dl >= 0, dl < P))
                dlvs[slot][pl.ds(16 * j, 16)] = jnp.where(ok, dl, TRASH) * 128

    def fire_g(c, slot):
        @pl.when(c < K)
        def _():
            pltpu.async_copy(feat_hbm.at[sidxs[slot]],
                             rowss[slot], sems_g[slot])

    def drain_g(c, slot):
        @pl.when(c < K)
        def _():
            pltpu.make_async_copy(feat_hbm.at[sidxs[slot]],
                                  rowss[slot], sems_g[slot]).wait()

    def one_edge(slot, j):
        dlb = plsc.load_gather(dlvs[slot], [jnp.full((16,), j, jnp.int32)])
        abase = dlb + iota
        addrs = [abase + (16 * v) for v in range(8)]
        accs = [plsc.load_gather(acc, [a]) for a in addrs]
        rws = [rowss[slot][j, pl.ds(16 * v, 16)] for v in range(8)]
        mx = [jnp.maximum(a, r) for a, r in zip(accs, rws)]
        for a, m in zip(addrs, mx):
            plsc.store_scatter(acc, [a], m)

    def hotloop(c, slot):
        @pl.when(c < K)
        def _():
            @pl.loop(0, CHUNK, step=4)
            def _(j):
                one_edge(slot, j)
                one_edge(slot, j + 1)
                one_edge(slot, j + 2)
                one_edge(slot, j + 3)

    def ibody(i, z):
        c0 = i * NB
        for k in range(NB):
            fire_e(c0 + k, k)
        for k in range(NB):
            decode(c0 + k, k)
        for k in range(NB):
            fire_g(c0 + k, k)
        for k in range(NB):
            drain_g(c0 + k, k)
        for k in range(NB):
            hotloop(c0 + k, k)
        return z

    lax.fori_loop(0, lax.div(K + (NB - 1), NB), ibody, 0)

    pltpu.async_copy(acc.at[pl.ds(0, P * D)],
                     agg_hbm.at[pl.ds(blo * D, P * D)], sco).wait()


@jax.jit
def _aggregate(feat, edges, offs):
    kern = pl.kernel(
        _agg_body,
        out_type=jax.ShapeDtypeStruct((NPAD * D,), jnp.float32),
        mesh=_mesh,
        compiler_params=_sc_params,
        scratch_types=[
            pltpu.VMEM((ROWPAD * D,), jnp.float32),
            pltpu.VMEM((4, CHUNK, D), jnp.float32),
            pltpu.VMEM((4, CHUNK), jnp.int32),
            pltpu.VMEM((4, CHUNK), jnp.int32),
            pltpu.VMEM((4, CHUNK), jnp.int32),
            pltpu.VMEM((NW * OFFW,), jnp.int32),
            pltpu.VMEM((KMAX,), jnp.int32),
            pltpu.VMEM((KMAX,), jnp.int32),
            pltpu.SemaphoreType.DMA((4,)),
            pltpu.SemaphoreType.DMA((4,)),
            pltpu.SemaphoreType.DMA,
        ],
    )
    return kern(feat, edges, offs).reshape(NPAD, D)


_ROWS = 1000


def _matr_body(x_ref, wrT_ref, o_ref):
    o_ref[...] = jnp.dot(x_ref[...], wrT_ref[...],
                         preferred_element_type=jnp.float32)


def _matr(x, Wr):
    return pl.pallas_call(
        _matr_body,
        grid=(N // _ROWS,),
        in_specs=[
            pl.BlockSpec((_ROWS, D), lambda i: (i, 0)),
            pl.BlockSpec((D, D), lambda i: (0, 0)),
        ],
        out_specs=pl.BlockSpec((_ROWS, D), lambda i: (i, 0)),
        out_shape=jax.ShapeDtypeStruct((N, D), jnp.float32),
    )(x, Wr.T)


def _dense_body(a_ref, xr_ref, wlT_ref, bl_ref, o_ref):
    a = a_ref[...]
    a = jnp.where(jnp.isneginf(a), 0.0, a)
    acc = jnp.dot(a, wlT_ref[...], preferred_element_type=jnp.float32)
    acc += xr_ref[...]
    acc += bl_ref[...]
    o_ref[...] = jnp.where(acc >= 0, acc, 0.01 * acc)


def _dense(agg, xr, Wl, bl):
    grid = (N // _ROWS,)
    return pl.pallas_call(
        _dense_body,
        grid=grid,
        in_specs=[
            pl.BlockSpec((_ROWS, D), lambda i: (i, 0)),
            pl.BlockSpec((_ROWS, D), lambda i: (i, 0)),
            pl.BlockSpec((D, D), lambda i: (0, 0)),
            pl.BlockSpec((1, D), lambda i: (0, 0)),
        ],
        out_specs=pl.BlockSpec((_ROWS, D), lambda i: (i, 0)),
        out_shape=jax.ShapeDtypeStruct((N, D), jnp.float32),
    )(agg, xr, Wl.T, bl.reshape(1, D))


def kernel(x, edge_index, Wl1, bl1, Wr1, Wl2, bl2, Wr2):
    edges, offs = _route(edge_index)
    agg1 = _aggregate(x, edges, offs)
    xr1 = _matr(x, Wr1)
    h = _dense(agg1, xr1, Wl1, bl1)
    agg2 = _aggregate(h, edges, offs)
    xr2 = _matr(h, Wr2)
    return _dense(agg2, xr2, Wl2, bl2)
